# Initial kernel scaffold; baseline (speedup 1.0000x reference)
import jax
import jax.numpy as jnp
from jax.experimental import pallas as pl


def kernel(h, edge_index, e, W_fc, W_attn, W_edge, W_ez, Wm1, Wm2):
    raise NotImplementedError("write your pallas kernel here")


# speedup vs baseline: 7.4009x; 7.4009x over previous
import functools

import jax
import jax.numpy as jnp
from jax import lax
from jax.experimental import pallas as pl
from jax.experimental.pallas import tpu as pltpu
from jax.experimental.pallas import tpu_sc as plsc

N = 10000
E = 320000
D = 128
ED = 16

NP = 10240
EP = 327680
NC = 2
NS = 16
CW = EP // (NC * NS * 128)
STRIPE = NP // NS


def _leaky(x):
    return jnp.where(x >= 0, x, 0.01 * x)


def _sc_counts(src2d, dst2d):
    mesh = plsc.VectorSubcoreMesh(core_axis_name="c", subcore_axis_name="s")

    @functools.partial(
        pl.kernel,
        out_type=[jax.ShapeDtypeStruct((NC * NP,), jnp.float32),
                  jax.ShapeDtypeStruct((NC * NP,), jnp.float32)],
        mesh=mesh,
        compiler_params=pltpu.CompilerParams(needs_layout_passes=False, use_tc_tiling_on_sc=False),
        scratch_types=[pltpu.VMEM((CW, 128), jnp.int32),
                       pltpu.VMEM((CW, 128), jnp.int32),
                       pltpu.VMEM((128,), jnp.float32),
                       pltpu.VMEM((STRIPE,), jnp.float32),
                       pltpu.VMEM_SHARED((NP,), jnp.float32),
                       pltpu.VMEM_SHARED((NP,), jnp.float32)],
    )
    def body(src_hbm, dst_hbm, out_s, out_d, src_v, dst_v, ones_v, zero_v,
             cs_sp, cd_sp):
        c = lax.axis_index("c")
        s = lax.axis_index("s")
        wid = c * NS + s
        for k in range(8):
            ones_v[pl.ds(k * 16, 16)] = jnp.full((16,), 1.0, jnp.float32)

        def zloop(i, _):
            zero_v[pl.ds(i * 16, 16)] = jnp.zeros((16,), jnp.float32)
            return 0
        lax.fori_loop(0, STRIPE // 16, zloop, 0)
        pltpu.sync_copy(src_hbm.at[pl.ds(wid * CW, CW)], src_v)
        pltpu.sync_copy(dst_hbm.at[pl.ds(wid * CW, CW)], dst_v)
        pltpu.sync_copy(zero_v, cs_sp.at[pl.ds(s * STRIPE, STRIPE)])
        pltpu.sync_copy(zero_v, cd_sp.at[pl.ds(s * STRIPE, STRIPE)])
        plsc.subcore_barrier()

        def chunk(j, _):
            pltpu.sync_copy(ones_v, cs_sp.at[src_v.at[j]], add=True)
            pltpu.sync_copy(ones_v, cd_sp.at[dst_v.at[j]], add=True)
            return 0
        lax.fori_loop(0, CW, chunk, 0)
        plsc.subcore_barrier()
        base = c * NP + s * STRIPE
        pltpu.sync_copy(cs_sp.at[pl.ds(s * STRIPE, STRIPE)],
                        out_s.at[pl.ds(base, STRIPE)])
        pltpu.sync_copy(cd_sp.at[pl.ds(s * STRIPE, STRIPE)],
                        out_d.at[pl.ds(base, STRIPE)])

    return body(src2d, dst2d)


def _tc_project(h_pad, W_fc, U2, counts2):
    R = 1024
    grid = NP // R

    def body(h_ref, w_ref, u2_ref, cnt_ref, z_ref, pq_ref, s_ref):
        z = lax.dot_general(h_ref[...], w_ref[...], (((1,), (1,)), ((), ())),
                            preferred_element_type=jnp.float32)
        z_ref[...] = z
        pq_ref[...] = lax.dot_general(u2_ref[...], z, (((1,), (1,)), ((), ())),
                                      preferred_element_type=jnp.float32)
        cnt = cnt_ref[...]
        z2 = z * z
        z3 = z2 * z
        s1 = lax.dot_general(cnt, z, (((1,), (0,)), ((), ())),
                             preferred_element_type=jnp.float32)
        s2 = lax.dot_general(cnt, z2, (((1,), (0,)), ((), ())),
                             preferred_element_type=jnp.float32)
        s3 = lax.dot_general(cnt, z3, (((1,), (0,)), ((), ())),
                             preferred_element_type=jnp.float32)
        scat = jnp.concatenate([s1, s2, s3, jnp.zeros((2, 128), jnp.float32)], 0)

        @pl.when(pl.program_id(0) == 0)
        def _():
            s_ref[...] = scat

        @pl.when(pl.program_id(0) != 0)
        def _():
            s_ref[...] = s_ref[...] + scat

    return pl.pallas_call(
        body,
        grid=(grid,),
        in_specs=[pl.BlockSpec((R, D), lambda i: (i, 0)),
                  pl.BlockSpec((D, D), lambda i: (0, 0)),
                  pl.BlockSpec((2, D), lambda i: (0, 0)),
                  pl.BlockSpec((2, R), lambda i: (0, i))],
        out_specs=[pl.BlockSpec((R, D), lambda i: (i, 0)),
                   pl.BlockSpec((2, R), lambda i: (0, i)),
                   pl.BlockSpec((8, D), lambda i: (0, 0))],
        out_shape=[jax.ShapeDtypeStruct((NP, D), jnp.float32),
                   jax.ShapeDtypeStruct((2, NP), jnp.float32),
                   jax.ShapeDtypeStruct((8, D), jnp.float32)],
        compiler_params=pltpu.CompilerParams(
            dimension_semantics=("arbitrary",)),
    )(h_pad, W_fc, U2, counts2)


def _tc_logit_const(pq, S, A4, Wm1, Wm2):
    def body(pq_ref, s_ref, a4_ref, wm1_ref, wm2_ref, out_ref):
        Ef = jnp.float32(E)
        sall = s_ref[...]

        def cpart(s1, s2, s3, a2, a3, wm1, wm2):
            mu = s1 / Ef
            m2 = s2 / Ef - mu * mu
            m3 = s3 / Ef - 3.0 * mu * (s2 / Ef) + 2.0 * mu * mu * mu
            r2 = jnp.sign(m2) * jnp.sqrt(jnp.abs(m2))
            r3 = jnp.sign(m3) * jnp.exp(jnp.log(jnp.abs(m3)) * (1.0 / 3.0))
            t2 = lax.dot_general(r2, wm1, (((1,), (1,)), ((), ())),
                                 preferred_element_type=jnp.float32)
            t3 = lax.dot_general(r3, wm2, (((1,), (1,)), ((), ())),
                                 preferred_element_type=jnp.float32)
            return jnp.sum(t2 * a2) + jnp.sum(t3 * a3)

        a4 = a4_ref[...]
        wm1 = wm1_ref[...]
        wm2 = wm2_ref[...]
        cs = cpart(sall[0:1], sall[2:3], sall[4:5], a4[0:1], a4[1:2], wm1, wm2)
        cd = cpart(sall[1:2], sall[3:4], sall[5:6], a4[2:3], a4[3:4], wm1, wm2)
        C = cs + cd
        pqv = pq_ref[...]
        out_ref[...] = jnp.concatenate([pqv[0:1] + C, pqv[1:2]], 0)

    return pl.pallas_call(
        body,
        out_shape=jax.ShapeDtypeStruct((2, NP), jnp.float32),
    )(pq, S, A4, Wm1, Wm2)


def _tc_edge_logit(e_pad, W_edge, ae):
    R = 2048
    grid = EP // R

    def body(e_ref, we_ref, ae_ref, r_ref):
        wv = lax.dot_general(ae_ref[...], we_ref[...], (((1,), (0,)), ((), ())),
                             preferred_element_type=jnp.float32)
        r_ref[...] = lax.dot_general(wv, e_ref[...], (((1,), (1,)), ((), ())),
                                     preferred_element_type=jnp.float32)

    return pl.pallas_call(
        body,
        grid=(grid,),
        in_specs=[pl.BlockSpec((R, ED), lambda i: (i, 0)),
                  pl.BlockSpec((ED, ED), lambda i: (0, 0)),
                  pl.BlockSpec((1, ED), lambda i: (0, 0))],
        out_specs=pl.BlockSpec((1, R), lambda i: (0, i)),
        out_shape=jax.ShapeDtypeStruct((1, EP), jnp.float32),
    )(e_pad, W_edge, ae)


def _sc_softmax_num(src2d, dst2d, r2d, pq2):
    mesh = plsc.VectorSubcoreMesh(core_axis_name="c", subcore_axis_name="s")

    @functools.partial(
        pl.kernel,
        out_type=[jax.ShapeDtypeStruct((EP // 128, 128), jnp.float32),
                  jax.ShapeDtypeStruct((NC * NP,), jnp.float32)],
        mesh=mesh,
        compiler_params=pltpu.CompilerParams(needs_layout_passes=False, use_tc_tiling_on_sc=False),
        scratch_types=[pltpu.VMEM((NP,), jnp.float32),
                       pltpu.VMEM((NP,), jnp.float32),
                       pltpu.VMEM((CW, 128), jnp.int32),
                       pltpu.VMEM((CW, 128), jnp.int32),
                       pltpu.VMEM((CW, 128), jnp.float32),
                       pltpu.VMEM((CW, 128), jnp.float32),
                       pltpu.VMEM((STRIPE,), jnp.float32),
                       pltpu.VMEM_SHARED((NP,), jnp.float32)],
    )
    def body(src_hbm, dst_hbm, r_hbm, pq_hbm, w_out, den_out,
             p_v, q_v, src_v, dst_v, r_v, w_v, zero_v, den_sp):
        c = lax.axis_index("c")
        s = lax.axis_index("s")
        wid = c * NS + s
        pltpu.sync_copy(pq_hbm.at[0], p_v)
        pltpu.sync_copy(pq_hbm.at[1], q_v)
        pltpu.sync_copy(src_hbm.at[pl.ds(wid * CW, CW)], src_v)
        pltpu.sync_copy(dst_hbm.at[pl.ds(wid * CW, CW)], dst_v)
        pltpu.sync_copy(r_hbm.at[pl.ds(wid * CW, CW)], r_v)

        def zloop(i, _):
            zero_v[pl.ds(i * 16, 16)] = jnp.zeros((16,), jnp.float32)
            return 0
        lax.fori_loop(0, STRIPE // 16, zloop, 0)
        pltpu.sync_copy(zero_v, den_sp.at[pl.ds(s * STRIPE, STRIPE)])
        plsc.subcore_barrier()

        def row(j, _):
            for k in range(8):
                sl = pl.ds(k * 16, 16)
                ps = plsc.load_gather(p_v, [src_v[j, sl]])
                qs = plsc.load_gather(q_v, [dst_v[j, sl]])
                logit = _leaky(ps + qs + r_v[j, sl])
                w_v[j, sl] = jnp.exp(logit)
            return 0
        lax.fori_loop(0, CW, row, 0)

        def chunk(j, _):
            pltpu.sync_copy(w_v.at[j], den_sp.at[dst_v.at[j]], add=True)
            return 0
        lax.fori_loop(0, CW, chunk, 0)
        pltpu.sync_copy(w_v, w_out.at[pl.ds(wid * CW, CW)])
        plsc.subcore_barrier()
        base = c * NP + s * STRIPE
        pltpu.sync_copy(den_sp.at[pl.ds(s * STRIPE, STRIPE)],
                        den_out.at[pl.ds(base, STRIPE)])

    return body(src2d, dst2d, r2d, pq2)


def _tc_scale_e(e_pad, w1):
    R = 2048
    grid = EP // R

    def body(e_ref, w_ref, ew_ref):
        eye = jnp.eye(ED, dtype=jnp.float32)
        et = lax.dot_general(eye, e_ref[...], (((1,), (1,)), ((), ())),
                             preferred_element_type=jnp.float32)
        ew_ref[...] = et * w_ref[...]

    return pl.pallas_call(
        body,
        grid=(grid,),
        in_specs=[pl.BlockSpec((R, ED), lambda i: (i, 0)),
                  pl.BlockSpec((1, R), lambda i: (0, i))],
        out_specs=pl.BlockSpec((ED, R), lambda i: (0, i)),
        out_shape=jax.ShapeDtypeStruct((ED, EP), jnp.float32),
    )(e_pad, w1)


HD = D // 2
HE = ED // 2
CWF = EP // (NS * 128)


def _sc_weighted_scatter(src2d, dst2d, w2d, zz, ew):
    mesh = plsc.VectorSubcoreMesh(core_axis_name="c", subcore_axis_name="s")

    @functools.partial(
        pl.kernel,
        out_type=[jax.ShapeDtypeStruct((NC * NP, HD), jnp.float32)]
                 + [jax.ShapeDtypeStruct((NC * NP,), jnp.float32)
                    for _ in range(HE)],
        mesh=mesh,
        compiler_params=pltpu.CompilerParams(needs_layout_passes=False, use_tc_tiling_on_sc=False),
        scratch_types=[pltpu.VMEM((CWF, 128), jnp.int32),
                       pltpu.VMEM((CWF, 128), jnp.int32),
                       pltpu.VMEM((CWF, 128), jnp.float32),
                       pltpu.VMEM((128, HD), jnp.float32),
                       pltpu.VMEM((HE, 128), jnp.float32),
                       pltpu.VMEM((STRIPE,), jnp.float32),
                       pltpu.VMEM_SHARED((NP, HD), jnp.float32)]
                      + [pltpu.VMEM_SHARED((NP,), jnp.float32)
                         for _ in range(HE)],
    )
    def body(src_hbm, dst_hbm, w_hbm, zz_hbm, ew_hbm, *rest):
        acc_out = rest[0]
        f_outs = rest[1:1 + HE]
        src_v, dst_v, w_v, zbuf, ewbuf, zero1 = rest[1 + HE:7 + HE]
        acc_sp = rest[7 + HE]
        f_sps = rest[8 + HE:8 + HE + HE]
        c = lax.axis_index("c")
        s = lax.axis_index("s")
        pltpu.sync_copy(src_hbm.at[pl.ds(s * CWF, CWF)], src_v)
        pltpu.sync_copy(dst_hbm.at[pl.ds(s * CWF, CWF)], dst_v)
        pltpu.sync_copy(w_hbm.at[pl.ds(s * CWF, CWF)], w_v)
        roff = jnp.full((16,), c * NP, jnp.int32)

        def offs(j, _):
            for g in range(8):
                sl = pl.ds(g * 16, 16)
                src_v[j, sl] = src_v[j, sl] + roff
            return 0
        lax.fori_loop(0, CWF, offs, 0)

        def zb(i, _):
            for k in range(HD // 16):
                zbuf[i, pl.ds(k * 16, 16)] = jnp.zeros((16,), jnp.float32)
            return 0
        lax.fori_loop(0, 128, zb, 0)

        def z1(i, _):
            zero1[pl.ds(i * 16, 16)] = jnp.zeros((16,), jnp.float32)
            return 0
        lax.fori_loop(0, STRIPE // 16, z1, 0)
        for t in range(STRIPE // 128):
            pltpu.sync_copy(zbuf, acc_sp.at[pl.ds(s * STRIPE + t * 128, 128)])
        for k in range(HE):
            pltpu.sync_copy(zero1, f_sps[k].at[pl.ds(s * STRIPE, STRIPE)])
        plsc.subcore_barrier()

        fro = c * HE

        def chunk(j, _):
            pltpu.sync_copy(zz_hbm.at[src_v.at[j]], zbuf)
            pltpu.sync_copy(
                ew_hbm.at[pl.ds(fro, HE), pl.ds((s * CWF + j) * 128, 128)],
                ewbuf)

            def groupscale(g, _):
                wg = w_v[j, pl.ds(g * 16, 16)]
                for l in range(16):
                    i = g * 16 + l
                    wb = jnp.full((16,), wg[l], jnp.float32)
                    for k in range(HD // 16):
                        sl = pl.ds(k * 16, 16)
                        zbuf[i, sl] = zbuf[i, sl] * wb
                return 0
            lax.fori_loop(0, 8, groupscale, 0)
            pltpu.sync_copy(zbuf, acc_sp.at[dst_v.at[j]], add=True)
            for k in range(HE):
                pltpu.sync_copy(ewbuf.at[k], f_sps[k].at[dst_v.at[j]],
                                add=True)
            return 0
        lax.fori_loop(0, CWF, chunk, 0)
        plsc.subcore_barrier()
        base = c * NP + s * STRIPE
        pltpu.sync_copy(acc_sp.at[pl.ds(s * STRIPE, STRIPE)],
                        acc_out.at[pl.ds(base, STRIPE)])
        for k in range(HE):
            pltpu.sync_copy(f_sps[k].at[pl.ds(s * STRIPE, STRIPE)],
                            f_outs[k].at[pl.ds(base, STRIPE)])

    return body(src2d, dst2d, w2d, zz, ew)


def _tc_combine(acc_parts, acc16_parts, den_col, W_ez, W_edge):
    R = 1024
    grid = NP // R

    def body(a_ref, a16_ref, d_ref, wez_ref, wedge_ref, out_ref):
        A = a_ref[...]
        B16 = a16_ref[...]
        WW = lax.dot_general(wez_ref[...], wedge_ref[...],
                             (((1,), (0,)), ((), ())),
                             preferred_element_type=jnp.float32)
        contrib = lax.dot_general(B16, WW, (((1,), (1,)), ((), ())),
                                  preferred_element_type=jnp.float32)
        d = d_ref[...]
        d = jnp.where(d > 0, d, 1.0)
        out_ref[...] = (A + contrib) / d

    return pl.pallas_call(
        body,
        grid=(grid,),
        in_specs=[pl.BlockSpec((R, D), lambda i: (i, 0)),
                  pl.BlockSpec((R, ED), lambda i: (i, 0)),
                  pl.BlockSpec((R, 1), lambda i: (i, 0)),
                  pl.BlockSpec((D, ED), lambda i: (0, 0)),
                  pl.BlockSpec((ED, ED), lambda i: (0, 0))],
        out_specs=pl.BlockSpec((R, D), lambda i: (i, 0)),
        out_shape=jax.ShapeDtypeStruct((NP, D), jnp.float32),
    )(acc_parts, acc16_parts, den_col, W_ez, W_edge)


def kernel(h, edge_index, e, W_fc, W_attn, W_edge, W_ez, Wm1, Wm2):
    f32 = jnp.float32
    h_pad = jnp.zeros((NP, D), f32).at[:N].set(h)
    src = edge_index[0]
    dst = edge_index[1]
    pad = jnp.full((EP - E,), N, jnp.int32)
    src_p = jnp.concatenate([src, pad]).reshape(EP // 128, 128)
    dst_p = jnp.concatenate([dst, pad]).reshape(EP // 128, 128)
    e_pad = jnp.zeros((EP, ED), f32).at[:E].set(e)

    a = W_attn[0]
    U2 = jnp.stack([a[0:D], a[3 * D:4 * D]])
    A4 = jnp.stack([a[D:2 * D], a[2 * D:3 * D],
                    a[4 * D:5 * D], a[5 * D:6 * D]])
    ae = a[6 * D:6 * D + ED][None, :]

    cnt_s, cnt_d = _sc_counts(src_p, dst_p)
    counts2 = jnp.stack([cnt_s[:NP] + cnt_s[NP:], cnt_d[:NP] + cnt_d[NP:]])

    z, pq, S = _tc_project(h_pad, W_fc, U2, counts2)

    pq2 = _tc_logit_const(pq, S, A4, Wm1, Wm2)

    r = _tc_edge_logit(e_pad, W_edge, ae).reshape(EP // 128, 128)

    w2d, den = _sc_softmax_num(src_p, dst_p, r, pq2)

    ew = _tc_scale_e(e_pad, w2d.reshape(1, EP))

    zz = jnp.concatenate([z[:, :HD], z[:, HD:]], axis=0)
    outs = _sc_weighted_scatter(src_p, dst_p, w2d, zz, ew)
    acc = jnp.concatenate([outs[0][:NP], outs[0][NP:]], axis=1)
    acc16 = jnp.stack(
        [outs[1 + (f % HE)][(f // HE) * NP:(f // HE + 1) * NP]
         for f in range(ED)], axis=1)
    den_col = (den[:NP] + den[NP:])[:, None]

    out = _tc_combine(acc, acc16, den_col, W_ez, W_edge)
    return out[:N]
